# baseline (speedup 1.0000x reference)
import functools
import math

import jax
import jax.numpy as jnp
from jax import lax
from jax.experimental import pallas as pl
from jax.experimental.pallas import tpu as pltpu
from jax.experimental.pallas import tpu_sc as plsc

N = 1024 * 200
D = 128
L = 16
NC, NS = 2, 16
NW = NC * NS
PER_W = N // NW
CHUNK = 128
NCH = PER_W // CHUNK
NPAIR = NCH // 2
DC = D // L


def _rsqrt(v):
    i = lax.bitcast_convert_type(v, jnp.int32)
    y = lax.bitcast_convert_type(jnp.int32(0x5F3759DF) - (i >> 1), jnp.float32)
    h = 0.5 * v
    for _ in range(2):
        y = y * (1.5 - h * y * y)
    return y


def _tree(vs):
    while len(vs) > 1:
        nxt = [vs[i] + vs[i + 1] for i in range(0, len(vs) - 1, 2)]
        if len(vs) % 2:
            nxt[-1] = nxt[-1] + vs[-1]
        vs = nxt
    return vs[0]


def _body(ids, tids, dids, wq, wc, g, b, out,
          idb, cib, ddb, xb, sb, wcv, gv, bv, mvb, ivb,
          sem_g0, sem_g1, sem_s0, sem_s1, sem_o0, sem_o1):
    wid = lax.axis_index("s") * NC + lax.axis_index("c")
    base = wid * PER_W
    sems_g = (sem_g0, sem_g1)
    sems_s = (sem_s0, sem_s1)
    sems_o = (sem_o0, sem_o1)

    pltpu.sync_copy(g, gv)
    pltpu.sync_copy(b, bv)

    @pl.when(lax.axis_index("s") == 0)
    def _():
        pltpu.sync_copy(wc, wcv)
    plsc.subcore_barrier()

    pltpu.sync_copy(ids.at[pl.ds(base, PER_W)], idb)
    pltpu.sync_copy(tids.at[pl.ds(base, PER_W)], cib)
    pltpu.sync_copy(dids.at[pl.ds(base, PER_W)], ddb)

    @plsc.parallel_loop(0, PER_W // L, 1, unroll=8)
    def _ci(i):
        s = pl.ds(i * L, L)
        cib[s] = cib[s] * 16 + ddb[s]

    def issue_gathers(c, k):
        pltpu.async_copy(wq.at[idb.at[pl.ds(c * CHUNK, CHUNK)]],
                         xb.at[k], sems_g[k])
        pltpu.async_copy(wcv.at[cib.at[pl.ds(c * CHUNK, CHUNK)]],
                         sb.at[k], sems_s[k])

    def wait_gathers(k):
        pltpu.make_async_copy(wq.at[idb.at[pl.ds(0, CHUNK)]],
                              xb.at[k], sems_g[k]).wait()
        pltpu.make_async_copy(wcv.at[cib.at[pl.ds(0, CHUNK)]],
                              sb.at[k], sems_s[k]).wait()

    def wait_out(k):
        pltpu.make_async_copy(xb.at[k], out.at[pl.ds(base, CHUNK), :],
                              sems_o[k]).wait()

    issue_gathers(0, 0)

    gvs = [gv[pl.ds(j * L, L)] for j in range(DC)]
    bvs = [bv[pl.ds(j * L, L)] for j in range(DC)]

    def compute_chunk(k):
        @plsc.parallel_loop(0, CHUNK, 1, unroll=4)
        def _stats(t):
            x0 = xb[k, t, pl.ds(0, L)] + sb[k, t, pl.ds(0, L)]
            xb[k, t, pl.ds(0, L)] = x0
            acc1 = x0
            acc2 = x0 * x0
            for j in range(1, DC):
                x = xb[k, t, pl.ds(j * L, L)] + sb[k, t, pl.ds(j * L, L)]
                xb[k, t, pl.ds(j * L, L)] = x
                acc1 = acc1 + x
                acc2 = acc2 + x * x
            mu = jnp.sum(acc1) * (1.0 / D)
            es2 = jnp.sum(acc2) * (1.0 / D)
            inv = _rsqrt(es2 - mu * mu + 1e-12)
            mvb[pl.ds(t * L, L)] = jnp.full((L,), mu)
            ivb[pl.ds(t * L, L)] = jnp.full((L,), inv)

        @plsc.parallel_loop(0, CHUNK, 1, unroll=4)
        def _norm(t):
            mu = mvb[pl.ds(t * L, L)]
            inv = ivb[pl.ds(t * L, L)]
            for j in range(DC):
                o = (xb[k, t, pl.ds(j * L, L)] - mu) * inv * gvs[j] + bvs[j]
                xb[k, t, pl.ds(j * L, L)] = o

    def pair_body(m, _):
        for k in (0, 1):
            c = 2 * m + k
            k2 = 1 - k
            if k == 0:
                @pl.when(m >= 1)
                def _():
                    wait_out(k2)
                issue_gathers(c + 1, k2)
            else:
                wait_out(k2)

                @pl.when(m < NPAIR - 1)
                def _():
                    issue_gathers(c + 1, k2)
            wait_gathers(k)
            compute_chunk(k)
            pltpu.async_copy(xb.at[k],
                             out.at[pl.ds(base + c * CHUNK, CHUNK), :],
                             sems_o[k])
        return 0

    lax.fori_loop(0, NPAIR, pair_body, 0)
    wait_out(1)


@jax.jit
def _sc_text_emb(ids, tids, dids, wq, wc, g, b):
    mesh = plsc.VectorSubcoreMesh(core_axis_name="c", subcore_axis_name="s")
    fn = pl.kernel(
        _body,
        out_type=jax.ShapeDtypeStruct((N, D), jnp.float32),
        mesh=mesh,
        compiler_params=pltpu.CompilerParams(needs_layout_passes=False),
        scratch_types=[
            pltpu.VMEM((PER_W,), jnp.int32),
            pltpu.VMEM((PER_W,), jnp.int32),
            pltpu.VMEM((PER_W,), jnp.int32),
            pltpu.VMEM((2, CHUNK, D), jnp.float32),
            pltpu.VMEM((2, CHUNK, D), jnp.float32),
            pltpu.VMEM_SHARED((112, D), jnp.float32),
            pltpu.VMEM((D,), jnp.float32),
            pltpu.VMEM((D,), jnp.float32),
            pltpu.VMEM((CHUNK * L,), jnp.float32),
            pltpu.VMEM((CHUNK * L,), jnp.float32),
            pltpu.SemaphoreType.DMA,
            pltpu.SemaphoreType.DMA,
            pltpu.SemaphoreType.DMA,
            pltpu.SemaphoreType.DMA,
            pltpu.SemaphoreType.DMA,
            pltpu.SemaphoreType.DMA,
        ],
    )
    return fn(ids, tids, dids, wq, wc, g, b)


def kernel(input_ids, type_ids, dpe_ids, W_input, W_type, W_dpe, pe,
           ln_gamma, ln_beta):
    B, S = input_ids.shape
    d = W_input.shape[1]
    ids = input_ids.reshape(-1).astype(jnp.int32)
    tid = type_ids.reshape(-1).astype(jnp.int32)
    did = dpe_ids.reshape(-1).astype(jnp.int32)
    wc = (W_type[:, None, :] + W_dpe[None, :, :]).reshape(-1, d) + pe[0, 0][None, :]
    out = _sc_text_emb(ids, tid, did, W_input, wc, ln_gamma, ln_beta)
    return out.reshape(B * S, 1, d)


# speedup vs baseline: 1.0036x; 1.0036x over previous
import functools
import math

import jax
import jax.numpy as jnp
from jax import lax
from jax.experimental import pallas as pl
from jax.experimental.pallas import tpu as pltpu
from jax.experimental.pallas import tpu_sc as plsc

N = 1024 * 200
D = 128
L = 16
NC, NS = 2, 16
NW = NC * NS
PER_W = N // NW
CHUNK = 128
NCH = PER_W // CHUNK
NPAIR = NCH // 2
DC = D // L


def _rsqrt(v):
    i = lax.bitcast_convert_type(v, jnp.int32)
    y = lax.bitcast_convert_type(jnp.int32(0x5F3759DF) - (i >> 1), jnp.float32)
    h = 0.5 * v
    for _ in range(2):
        y = y * (1.5 - h * y * y)
    return y


def _tree(vs):
    while len(vs) > 1:
        nxt = [vs[i] + vs[i + 1] for i in range(0, len(vs) - 1, 2)]
        if len(vs) % 2:
            nxt[-1] = nxt[-1] + vs[-1]
        vs = nxt
    return vs[0]


def _body(ids, tids, dids, wq, wc, g, b, out,
          idb, cib, ddb, xb, sb, wcv, gv, bv, mvb, ivb,
          sem_g0, sem_g1, sem_s0, sem_s1, sem_o0, sem_o1):
    wid = lax.axis_index("s") * NC + lax.axis_index("c")
    base = wid * PER_W
    sems_g = (sem_g0, sem_g1)
    sems_s = (sem_s0, sem_s1)
    sems_o = (sem_o0, sem_o1)

    pltpu.sync_copy(g, gv)
    pltpu.sync_copy(b, bv)

    @pl.when(lax.axis_index("s") == 0)
    def _():
        pltpu.sync_copy(wc, wcv)
    plsc.subcore_barrier()

    pltpu.sync_copy(ids.at[pl.ds(base, PER_W)], idb)
    pltpu.sync_copy(tids.at[pl.ds(base, PER_W)], cib)
    pltpu.sync_copy(dids.at[pl.ds(base, PER_W)], ddb)

    @plsc.parallel_loop(0, PER_W // L, 1, unroll=8)
    def _ci(i):
        s = pl.ds(i * L, L)
        cib[s] = cib[s] * 16 + ddb[s]

    def issue_gathers(c, k):
        pltpu.async_copy(wq.at[idb.at[pl.ds(c * CHUNK, CHUNK)]],
                         xb.at[k], sems_g[k])
        pltpu.async_copy(wcv.at[cib.at[pl.ds(c * CHUNK, CHUNK)]],
                         sb.at[k], sems_s[k])

    def wait_gathers(k):
        pltpu.make_async_copy(wq.at[idb.at[pl.ds(0, CHUNK)]],
                              xb.at[k], sems_g[k]).wait()
        pltpu.make_async_copy(wcv.at[cib.at[pl.ds(0, CHUNK)]],
                              sb.at[k], sems_s[k]).wait()

    def wait_out(k):
        pltpu.make_async_copy(xb.at[k], out.at[pl.ds(base, CHUNK), :],
                              sems_o[k]).wait()

    issue_gathers(0, 0)

    gvs = [gv[pl.ds(j * L, L)] for j in range(DC)]
    bvs = [bv[pl.ds(j * L, L)] for j in range(DC)]

    def compute_chunk(k):
        @plsc.parallel_loop(0, CHUNK, 1, unroll=4)
        def _stats(t):
            x0 = xb[k, t, pl.ds(0, L)] + sb[k, t, pl.ds(0, L)]
            xb[k, t, pl.ds(0, L)] = x0
            acc1 = x0
            acc2 = x0 * x0
            for j in range(1, DC):
                x = xb[k, t, pl.ds(j * L, L)] + sb[k, t, pl.ds(j * L, L)]
                xb[k, t, pl.ds(j * L, L)] = x
                acc1 = acc1 + x
                acc2 = acc2 + x * x
            mu = jnp.full((L,), jnp.sum(acc1)) * (1.0 / D)
            es2 = jnp.full((L,), jnp.sum(acc2)) * (1.0 / D)
            mvb[pl.ds(t * L, L)] = mu
            ivb[pl.ds(t * L, L)] = _rsqrt(es2 - mu * mu + 1e-12)

        @plsc.parallel_loop(0, CHUNK, 1, unroll=4)
        def _norm(t):
            mu = mvb[pl.ds(t * L, L)]
            inv = ivb[pl.ds(t * L, L)]
            for j in range(DC):
                o = (xb[k, t, pl.ds(j * L, L)] - mu) * inv * gvs[j] + bvs[j]
                xb[k, t, pl.ds(j * L, L)] = o

    def pair_body(m, _):
        for k in (0, 1):
            c = 2 * m + k
            k2 = 1 - k
            if k == 0:
                @pl.when(m >= 1)
                def _():
                    wait_out(k2)
                issue_gathers(c + 1, k2)
            else:
                wait_out(k2)

                @pl.when(m < NPAIR - 1)
                def _():
                    issue_gathers(c + 1, k2)
            wait_gathers(k)
            compute_chunk(k)
            pltpu.async_copy(xb.at[k],
                             out.at[pl.ds(base + c * CHUNK, CHUNK), :],
                             sems_o[k])
        return 0

    lax.fori_loop(0, NPAIR, pair_body, 0)
    wait_out(1)


@jax.jit
def _sc_text_emb(ids, tids, dids, wq, wc, g, b):
    mesh = plsc.VectorSubcoreMesh(core_axis_name="c", subcore_axis_name="s")
    fn = pl.kernel(
        _body,
        out_type=jax.ShapeDtypeStruct((N, D), jnp.float32),
        mesh=mesh,
        compiler_params=pltpu.CompilerParams(needs_layout_passes=False),
        scratch_types=[
            pltpu.VMEM((PER_W,), jnp.int32),
            pltpu.VMEM((PER_W,), jnp.int32),
            pltpu.VMEM((PER_W,), jnp.int32),
            pltpu.VMEM((2, CHUNK, D), jnp.float32),
            pltpu.VMEM((2, CHUNK, D), jnp.float32),
            pltpu.VMEM_SHARED((112, D), jnp.float32),
            pltpu.VMEM((D,), jnp.float32),
            pltpu.VMEM((D,), jnp.float32),
            pltpu.VMEM((CHUNK * L,), jnp.float32),
            pltpu.VMEM((CHUNK * L,), jnp.float32),
            pltpu.SemaphoreType.DMA,
            pltpu.SemaphoreType.DMA,
            pltpu.SemaphoreType.DMA,
            pltpu.SemaphoreType.DMA,
            pltpu.SemaphoreType.DMA,
            pltpu.SemaphoreType.DMA,
        ],
    )
    return fn(ids, tids, dids, wq, wc, g, b)


def kernel(input_ids, type_ids, dpe_ids, W_input, W_type, W_dpe, pe,
           ln_gamma, ln_beta):
    B, S = input_ids.shape
    d = W_input.shape[1]
    ids = input_ids.reshape(-1).astype(jnp.int32)
    tid = type_ids.reshape(-1).astype(jnp.int32)
    did = dpe_ids.reshape(-1).astype(jnp.int32)
    wc = (W_type[:, None, :] + W_dpe[None, :, :]).reshape(-1, d) + pe[0, 0][None, :]
    out = _sc_text_emb(ids, tid, did, W_input, wc, ln_gamma, ln_beta)
    return out.reshape(B * S, 1, d)


# speedup vs baseline: 1.0055x; 1.0019x over previous
import jax
import jax.numpy as jnp
from jax import lax
from jax.experimental import pallas as pl
from jax.experimental.pallas import tpu as pltpu
from jax.experimental.pallas import tpu_sc as plsc

N = 1024 * 200
D = 128
L = 16
NC, NS = 2, 16
NW = NC * NS
PER_W = N // NW
CHUNK = 128
NCH = PER_W // CHUNK
NPAIR = NCH // 2
DC = D // L


def _rsqrt(v):
    i = lax.bitcast_convert_type(v, jnp.int32)
    y = lax.bitcast_convert_type(jnp.int32(0x5F3759DF) - (i >> 1), jnp.float32)
    h = 0.5 * v
    for _ in range(2):
        y = y * (1.5 - h * y * y)
    return y


def _body(ids, tids, dids, wq, wc, g, b, out,
          idb, cib, ddb, xb, sb, wcv, gv, bv, mvb, ivb,
          sem_g0, sem_g1, sem_s0, sem_s1, sem_o0, sem_o1):
    wid = lax.axis_index("s") * NC + lax.axis_index("c")
    base = wid * PER_W
    sems_g = (sem_g0, sem_g1)
    sems_s = (sem_s0, sem_s1)
    sems_o = (sem_o0, sem_o1)

    pltpu.sync_copy(g, gv)
    pltpu.sync_copy(b, bv)

    @pl.when(lax.axis_index("s") == 0)
    def _():
        pltpu.sync_copy(wc, wcv)
    plsc.subcore_barrier()

    pltpu.sync_copy(ids.at[pl.ds(base, PER_W)], idb)
    pltpu.sync_copy(tids.at[pl.ds(base, PER_W)], cib)
    pltpu.sync_copy(dids.at[pl.ds(base, PER_W)], ddb)

    @plsc.parallel_loop(0, PER_W // L, 1, unroll=8)
    def _ci(i):
        s = pl.ds(i * L, L)
        cib[s] = cib[s] * 16 + ddb[s]

    def issue_gathers(c, k):
        pltpu.async_copy(wq.at[idb.at[pl.ds(c * CHUNK, CHUNK)]],
                         xb.at[k], sems_g[k])
        pltpu.async_copy(wcv.at[cib.at[pl.ds(c * CHUNK, CHUNK)]],
                         sb.at[k], sems_s[k])

    def wait_gathers(k):
        pltpu.make_async_copy(wq.at[idb.at[pl.ds(0, CHUNK)]],
                              xb.at[k], sems_g[k]).wait()
        pltpu.make_async_copy(wcv.at[cib.at[pl.ds(0, CHUNK)]],
                              sb.at[k], sems_s[k]).wait()

    def wait_out(k):
        pltpu.make_async_copy(xb.at[k], out.at[pl.ds(base, CHUNK), :],
                              sems_o[k]).wait()

    issue_gathers(0, 0)

    gvs = [gv[pl.ds(j * L, L)] for j in range(DC)]
    bvs = [bv[pl.ds(j * L, L)] for j in range(DC)]

    def compute_chunk(k):
        @plsc.parallel_loop(0, CHUNK, 1, unroll=4)
        def _stats(t):
            x0 = xb[k, t, pl.ds(0, L)] + sb[k, t, pl.ds(0, L)]
            xb[k, t, pl.ds(0, L)] = x0
            acc1 = x0
            acc2 = x0 * x0
            for j in range(1, DC):
                x = xb[k, t, pl.ds(j * L, L)] + sb[k, t, pl.ds(j * L, L)]
                xb[k, t, pl.ds(j * L, L)] = x
                acc1 = acc1 + x
                acc2 = acc2 + x * x
            mu = jnp.full((L,), jnp.sum(acc1)) * (1.0 / D)
            es2 = jnp.full((L,), jnp.sum(acc2)) * (1.0 / D)
            mvb[pl.ds(t * L, L)] = mu
            ivb[pl.ds(t * L, L)] = _rsqrt(es2 - mu * mu + 1e-12)

        @plsc.parallel_loop(0, CHUNK, 1, unroll=4)
        def _norm(t):
            mu = mvb[pl.ds(t * L, L)]
            inv = ivb[pl.ds(t * L, L)]
            for j in range(DC):
                o = (xb[k, t, pl.ds(j * L, L)] - mu) * inv * gvs[j] + bvs[j]
                xb[k, t, pl.ds(j * L, L)] = o

    def pair_body(m, _):
        for k in (0, 1):
            c = 2 * m + k
            k2 = 1 - k
            if k == 0:
                @pl.when(m >= 1)
                def _():
                    wait_out(k2)
                issue_gathers(c + 1, k2)
            else:
                wait_out(k2)

                @pl.when(m < NPAIR - 1)
                def _():
                    issue_gathers(c + 1, k2)
            wait_gathers(k)
            compute_chunk(k)
            pltpu.async_copy(xb.at[k],
                             out.at[pl.ds(base + c * CHUNK, CHUNK), :],
                             sems_o[k])
        return 0

    lax.fori_loop(0, NPAIR, pair_body, 0)
    wait_out(1)


@jax.jit
def _sc_text_emb(ids, tids, dids, wq, wc, g, b):
    mesh = plsc.VectorSubcoreMesh(core_axis_name="c", subcore_axis_name="s")
    fn = pl.kernel(
        _body,
        out_type=jax.ShapeDtypeStruct((N, D), jnp.float32),
        mesh=mesh,
        compiler_params=pltpu.CompilerParams(needs_layout_passes=False),
        scratch_types=[
            pltpu.VMEM((PER_W,), jnp.int32),
            pltpu.VMEM((PER_W,), jnp.int32),
            pltpu.VMEM((PER_W,), jnp.int32),
            pltpu.VMEM((2, CHUNK, D), jnp.float32),
            pltpu.VMEM((2, CHUNK, D), jnp.float32),
            pltpu.VMEM_SHARED((112, D), jnp.float32),
            pltpu.VMEM((D,), jnp.float32),
            pltpu.VMEM((D,), jnp.float32),
            pltpu.VMEM((CHUNK * L,), jnp.float32),
            pltpu.VMEM((CHUNK * L,), jnp.float32),
            pltpu.SemaphoreType.DMA,
            pltpu.SemaphoreType.DMA,
            pltpu.SemaphoreType.DMA,
            pltpu.SemaphoreType.DMA,
            pltpu.SemaphoreType.DMA,
            pltpu.SemaphoreType.DMA,
        ],
    )
    return fn(ids, tids, dids, wq, wc, g, b)


def kernel(input_ids, type_ids, dpe_ids, W_input, W_type, W_dpe, pe,
           ln_gamma, ln_beta):
    B, S = input_ids.shape
    d = W_input.shape[1]
    ids = input_ids.reshape(-1).astype(jnp.int32)
    tid = type_ids.reshape(-1).astype(jnp.int32)
    did = dpe_ids.reshape(-1).astype(jnp.int32)
    wc = (W_type[:, None, :] + W_dpe[None, :, :]).reshape(-1, d) + pe[0, 0][None, :]
    out = _sc_text_emb(ids, tid, did, W_input, wc, ln_gamma, ln_beta)
    return out.reshape(B * S, 1, d)
